# Initial kernel scaffold; baseline (speedup 1.0000x reference)
import jax
import jax.numpy as jnp
from jax.experimental import pallas as pl


def kernel(x, edge_index, batch, W1, b1, W2, b2, Wl, bl):
    raise NotImplementedError("write your pallas kernel here")


# speedup vs baseline: 20.2715x; 20.2715x over previous
import jax
import jax.numpy as jnp
from jax import lax
from jax.experimental import pallas as pl
from jax.experimental.pallas import tpu as pltpu
from jax.experimental.pallas import tpu_sc as plsc

N = 10000
E = 320000
F_IN = 128
H = 16
C = 10
G = 64

W = 128
NC = 2
NS = 16
NW = NC * NS
RW = 80
EPAD = NW * RW * 128
NPAD = 10112
RPS = NPAD // NS

_mesh = plsc.VectorSubcoreMesh(core_axis_name="c", subcore_axis_name="s")


def _agg_body(table_hbm, src_hbm, dst_hbm, zeros_hbm, out_hbm,
              sidx, didx, row, acc_sp):
    c = lax.axis_index("c")
    s = lax.axis_index("s")
    wid = c * NS + s
    pltpu.sync_copy(zeros_hbm.at[pl.ds(s * RPS, RPS)],
                    acc_sp.at[pl.ds(s * RPS, RPS)])
    pltpu.sync_copy(src_hbm.at[pl.ds(wid * RW, RW)], sidx)
    pltpu.sync_copy(dst_hbm.at[pl.ds(wid * RW, RW)], didx)
    plsc.subcore_barrier()

    def _step(j, carry):
        pltpu.sync_copy(table_hbm.at[sidx.at[j]], row)
        pltpu.sync_copy(row, acc_sp.at[didx.at[j]], add=True)
        return carry

    lax.fori_loop(0, RW, _step, 0)
    plsc.subcore_barrier()
    pltpu.sync_copy(acc_sp.at[pl.ds(s * RPS, RPS)],
                    out_hbm.at[c, pl.ds(s * RPS, RPS)])


_agg_call = pl.kernel(
    _agg_body,
    out_type=jax.ShapeDtypeStruct((NC, NPAD, W), jnp.float32),
    mesh=_mesh,
    scratch_types=[
        pltpu.VMEM((RW, 128), jnp.int32),
        pltpu.VMEM((RW, 128), jnp.int32),
        pltpu.VMEM((128, W), jnp.float32),
        pltpu.VMEM_SHARED((NPAD, W), jnp.float32),
    ],
)


def _deg_body(dst_hbm, ones_hbm, zeros_hbm, out_hbm, didx, ones_v, acc_sp):
    c = lax.axis_index("c")
    s = lax.axis_index("s")
    wid = c * NS + s
    pltpu.sync_copy(zeros_hbm.at[pl.ds(s * RPS, RPS)],
                    acc_sp.at[pl.ds(s * RPS, RPS)])
    pltpu.sync_copy(dst_hbm.at[pl.ds(wid * RW, RW)], didx)
    pltpu.sync_copy(ones_hbm, ones_v)
    plsc.subcore_barrier()

    def _step(j, carry):
        pltpu.sync_copy(ones_v, acc_sp.at[didx.at[j]], add=True)
        return carry

    lax.fori_loop(0, RW, _step, 0)
    plsc.subcore_barrier()
    pltpu.sync_copy(acc_sp.at[pl.ds(s * RPS, RPS)],
                    out_hbm.at[c, pl.ds(s * RPS, RPS)])


_deg_call = pl.kernel(
    _deg_body,
    out_type=jax.ShapeDtypeStruct((NC, NPAD, W), jnp.float32),
    mesh=_mesh,
    scratch_types=[
        pltpu.VMEM((RW, 128), jnp.int32),
        pltpu.VMEM((128, W), jnp.float32),
        pltpu.VMEM_SHARED((NPAD, W), jnp.float32),
    ],
)


def _tc_pre_body(x_ref, w1_ref, degp_ref, hn1_ref, dinv_ref):
    deg = degp_ref[0, :, 0:1] + degp_ref[1, :, 0:1] + 1.0
    dinv = lax.rsqrt(deg)
    h1 = jnp.dot(x_ref[...], w1_ref[...], preferred_element_type=jnp.float32)
    hn1_ref[...] = h1 * dinv
    dinv_ref[...] = jnp.broadcast_to(dinv, (NPAD, W))


_tc_pre = pl.pallas_call(
    _tc_pre_body,
    out_shape=[jax.ShapeDtypeStruct((NPAD, W), jnp.float32),
               jax.ShapeDtypeStruct((NPAD, W), jnp.float32)],
)


def _tc_mid_body(aggp_ref, hn1_ref, dinv_ref, b1_ref, w2_ref, hn2_ref):
    agg = aggp_ref[0] + aggp_ref[1] + hn1_ref[...]
    z = jnp.maximum(dinv_ref[...] * agg + b1_ref[...], 0.0)
    h2 = jnp.dot(z, w2_ref[...], preferred_element_type=jnp.float32)
    hn2_ref[...] = h2 * dinv_ref[...]


_tc_mid = pl.pallas_call(
    _tc_mid_body,
    out_shape=jax.ShapeDtypeStruct((NPAD, W), jnp.float32),
)


def _tc_post_body(aggp_ref, hn2_ref, dinv_ref, b2_ref, batch_ref, wl_ref,
                  bl_ref, out_ref):
    agg = aggp_ref[0] + aggp_ref[1] + hn2_ref[...]
    h2o = dinv_ref[...] * agg + b2_ref[...]
    gids = lax.broadcasted_iota(jnp.int32, (G, NPAD), 0)
    onehot = (gids == batch_ref[...]).astype(jnp.float32)
    sums = jnp.dot(onehot, h2o, preferred_element_type=jnp.float32)
    counts = jnp.sum(onehot, axis=1, keepdims=True)
    pooled = sums / jnp.maximum(counts, 1.0)
    logit = jnp.dot(pooled, wl_ref[...],
                    preferred_element_type=jnp.float32) + bl_ref[...]
    out_ref[...] = 1.0 / (1.0 + jnp.exp(-logit))


_tc_post = pl.pallas_call(
    _tc_post_body,
    out_shape=jax.ShapeDtypeStruct((G, 1), jnp.float32),
)


@jax.jit
def kernel(x, edge_index, batch, W1, b1, W2, b2, Wl, bl):
    src = edge_index[0]
    dst = edge_index[1]
    npadrows = NPAD - N
    pad_ids = (N + (jnp.arange(EPAD - E, dtype=jnp.int32) % npadrows))
    srcp = jnp.concatenate([src, pad_ids]).reshape(EPAD // 128, 128)
    dstp = jnp.concatenate([dst, pad_ids]).reshape(EPAD // 128, 128)
    xp = jnp.pad(x, ((0, npadrows), (0, 0)))
    batchp = jnp.pad(batch, (0, npadrows), constant_values=G).reshape(1, NPAD)
    w1p = jnp.pad(W1, ((0, 0), (0, W - H)))
    w2p = jnp.pad(W2, ((0, W - H), (0, W - C)))
    wlp = jnp.pad(Wl, ((0, W - C), (0, 0)))
    b1r = jnp.pad(b1, (0, W - H)).reshape(1, W)
    b2r = jnp.pad(b2, (0, W - C)).reshape(1, W)
    blr = bl.reshape(1, 1)
    zeros = jnp.zeros((NPAD, W), jnp.float32)
    ones3 = jnp.ones((128, W), jnp.float32)

    degp = _deg_call(dstp, ones3, zeros)
    hn1, dinv = _tc_pre(xp, w1p, degp)
    aggp1 = _agg_call(hn1, srcp, dstp, zeros)
    hn2 = _tc_mid(aggp1, hn1, dinv, b1r, w2p)
    aggp2 = _agg_call(hn2, srcp, dstp, zeros)
    return _tc_post(aggp2, hn2, dinv, b2r, batchp, wlp, blr)
